# Initial kernel scaffold; baseline (speedup 1.0000x reference)
import jax
import jax.numpy as jnp
from jax.experimental import pallas as pl


def kernel(x, training):
    raise NotImplementedError("write your pallas kernel here")


# speedup vs baseline: 2.0370x; 2.0370x over previous
import functools

import jax
import jax.numpy as jnp
import numpy as np
from jax import lax
from jax.experimental import pallas as pl
from jax.experimental.pallas import tpu as pltpu
from jax.experimental.pallas import tpu_sc as plsc

_FREQ_MASK = 10
_TIME_MASK = 20
_FREQ_MASK_RATE = 0.2
_TIME_MASK_RATE = 0.2

_NUM_CORES = 2
_NUM_SUBCORES = 16
_LANES = 16
_NW = _NUM_CORES * _NUM_SUBCORES


def _tf2x32(k1, k2, x1, x2):
    x = [np.uint32(x1), np.uint32(x2)]
    ks = [np.uint32(k1), np.uint32(k2),
          np.uint32(np.uint32(k1) ^ np.uint32(k2) ^ np.uint32(0x1BD11BDA))]
    rot = [(13, 15, 26, 6), (17, 29, 16, 24)]

    def _rl(v, d):
        return np.uint32((v << np.uint32(d)) | (v >> np.uint32(32 - d)))

    def _rounds(x, rs):
        for r in rs:
            x[0] = np.uint32(x[0] + x[1])
            x[1] = np.uint32(x[0] ^ _rl(x[1], r))
        return x

    with np.errstate(over="ignore"):
        x[0] = np.uint32(x[0] + ks[0])
        x[1] = np.uint32(x[1] + ks[1])
        for i in range(5):
            x = _rounds(x, rot[i % 2])
            x[0] = np.uint32(x[0] + ks[(i + 1) % 3])
            x[1] = np.uint32(x[1] + ks[(i + 2) % 3] + np.uint32(i + 1))
    return x[0], x[1]


def _np_split2(key):
    return (_tf2x32(key[0], key[1], 0, 0), _tf2x32(key[0], key[1], 0, 1))


def _np_randint(key, minval, maxval):
    k1, k2 = _np_split2(key)
    hb1, hb2 = _tf2x32(k1[0], k1[1], 0, 0)
    lb1, lb2 = _tf2x32(k2[0], k2[1], 0, 0)
    hi, lo = np.uint64(hb1 ^ hb2), np.uint64(lb1 ^ lb2)
    span = np.uint64(np.uint32(maxval - minval))
    mult = (np.uint64(65536) % span) ** 2 % span
    val = ((hi % span) * mult % span + lo % span) % span
    return int(minval + int(val))


@functools.lru_cache(maxsize=None)
def _mask_rows(t: int, f: int) -> tuple[int, ...]:
    f_mask = min(min(_FREQ_MASK, int(np.floor(float(f) * _FREQ_MASK_RATE))), f)
    t_mask = min(min(_TIME_MASK, int(np.floor(float(t) * _TIME_MASK_RATE))), t)
    key = (np.uint32(0), np.uint32(42))
    k1, k2 = _np_split2(key)
    f0 = _np_randint(k1, 0, max(1, f - f_mask))
    t0 = _np_randint(k2, 0, max(1, t - t_mask))
    rows = set(range(f0, f0 + f_mask)) | set(range(t0, t0 + t_mask))
    return tuple(sorted(r for r in rows if r < t))


@functools.lru_cache(maxsize=None)
def _build_sc_kernel(t: int, f: int, masked: tuple[int, ...]):
    assert t % 8 == 0, "row count must be a multiple of 8"
    groups = t // 8
    gbase, grem = divmod(groups, _NW)
    counts = [8 * (gbase + (1 if w < grem else 0)) for w in range(_NW)]
    starts = [0] * _NW
    for w in range(1, _NW):
        starts[w] = starts[w - 1] + counts[w - 1]
    maxc = max(counts)

    mesh = plsc.VectorSubcoreMesh(
        core_axis_name="c", subcore_axis_name="s",
        num_cores=_NUM_CORES, num_subcores=_NUM_SUBCORES)

    @functools.partial(
        pl.kernel,
        mesh=mesh,
        out_type=jax.ShapeDtypeStruct((t, f), jnp.float32),
        scratch_types=[
            pltpu.VMEM((maxc, f), jnp.float32),
            pltpu.VMEM((_LANES,), jnp.float32),
        ],
    )
    def sc_kernel(x_hbm, tv_hbm, out_hbm, buf_v, tv_v):
        wid = lax.axis_index("s") * _NUM_CORES + lax.axis_index("c")
        for w in range(_NW):
            s, n = starts[w], counts[w]
            rows_w = [r for r in masked if s <= r < s + n]

            @pl.when(wid == w)
            def _(s=s, n=n, rows_w=rows_w):
                pltpu.sync_copy(x_hbm.at[pl.ds(s, n)], buf_v.at[pl.ds(0, n)])
                if rows_w:
                    pltpu.sync_copy(tv_hbm, tv_v)
                    scale = tv_v[...]
                    for r in rows_w:
                        lr = r - s
                        for j in range(f // _LANES):
                            sl = pl.ds(j * _LANES, _LANES)
                            buf_v[lr, sl] = buf_v[lr, sl] * scale
                pltpu.sync_copy(buf_v.at[pl.ds(0, n)], out_hbm.at[pl.ds(s, n)])

    return sc_kernel


def kernel(x, training=True):
    t, f = x.shape
    masked = _mask_rows(t, f)
    sc_kernel = _build_sc_kernel(t, f, masked)
    keep = 1.0 - jnp.asarray(training, jnp.float32)
    tv = jnp.broadcast_to(keep, (_LANES,))
    return sc_kernel(x, tv)
